# baseline (speedup 1.0000x reference)
import functools

import jax
import jax.numpy as jnp
from jax import lax
from jax.experimental import pallas as pl
from jax.experimental.pallas import tpu as pltpu
from jax.experimental.pallas import tpu_sc as plsc

N_ROWS = 16384
N_CODES = 8192
DIM = 256
BETA_ = 1.0

BZ = 512
BW = 1024
NZ = N_ROWS // BZ
NWB = N_CODES // BW
LOSS_SCALE = (1.0 + BETA_) / (N_ROWS * DIM)


def _dist_argmin_body(z_ref, w_ref, idx_ref, loss_ref, runmin, runj,
                      wn_s, zn_s, w2_s):
    i = pl.program_id(0)
    j = pl.program_id(1)

    @pl.when(i == 0)
    def _():
        w = w_ref[pl.ds(j * BW, BW), :]
        w2_s[pl.ds(j * BW, BW), :] = w + w
        wn_s[:, pl.ds(j * BW, BW)] = jnp.sum(w * w, axis=1).reshape(1, BW)

    @pl.when(j == 0)
    def _():
        z0 = z_ref[...]
        zn_s[...] = jnp.sum(z0 * z0, axis=1, keepdims=True)

    mm2 = lax.dot_general(z_ref[...], w2_s[pl.ds(j * BW, BW), :],
                          dimension_numbers=(((1,), (1,)), ((), ())),
                          preferred_element_type=jnp.float32)
    dd = (zn_s[...] + wn_s[:, pl.ds(j * BW, BW)]) - mm2

    @pl.when(j == 0)
    def _():
        runmin[...] = dd
        runj[...] = jnp.zeros_like(runj)

    @pl.when(j > 0)
    def _():
        cur = runmin[...]
        upd = dd < cur
        runmin[...] = jnp.where(upd, dd, cur)
        runj[...] = jnp.where(upd, j, runj[...])

    @pl.when(j == NWB - 1)
    def _():
        rm = runmin[...]
        gmin = jnp.min(rm, axis=1)
        codes = runj[...] * BW + lax.broadcasted_iota(jnp.int32, rm.shape, 1)
        idx_ref[0, 0, :] = jnp.min(
            jnp.where(rm == gmin[:, None], codes, jnp.int32(2**31 - 1)), axis=1)
        part = jnp.sum(gmin).reshape(1, 1)
        prev = jnp.where(i == 0, jnp.zeros((1, 1), jnp.float32), loss_ref[...])
        tot = prev + part
        loss_ref[...] = jnp.where(i == NZ - 1, tot * LOSS_SCALE, tot)


def _dist_argmin(z, W):
    return pl.pallas_call(
        _dist_argmin_body,
        grid=(NZ, NWB),
        in_specs=[
            pl.BlockSpec((BZ, DIM), lambda i, j: (i, 0)),
            pl.BlockSpec((N_CODES, DIM), lambda i, j: (0, 0)),
        ],
        out_specs=[
            pl.BlockSpec((1, 1, BZ), lambda i, j: (i, 0, 0)),
            pl.BlockSpec((1, 1), lambda i, j: (0, 0)),
        ],
        out_shape=[
            jax.ShapeDtypeStruct((NZ, 1, BZ), jnp.int32),
            jax.ShapeDtypeStruct((1, 1), jnp.float32),
        ],
        scratch_shapes=[
            pltpu.VMEM((BZ, BW), jnp.float32),
            pltpu.VMEM((BZ, BW), jnp.int32),
            pltpu.VMEM((1, N_CODES), jnp.float32),
            pltpu.VMEM((BZ, 1), jnp.float32),
            pltpu.VMEM((N_CODES, DIM), jnp.float32),
        ],
        compiler_params=pltpu.CompilerParams(
            dimension_semantics=("arbitrary", "arbitrary")),
    )(z, W)


_NC = 2
_NS = 16
_NWK = _NC * _NS
_BPW = N_ROWS // _NWK
_CH = 128
_NCH = _BPW // _CH


def _sc_gather(W, idx):
    mesh = plsc.VectorSubcoreMesh(core_axis_name="c", subcore_axis_name="s")

    @functools.partial(
        pl.kernel, mesh=mesh,
        out_type=jax.ShapeDtypeStruct((N_ROWS, DIM), jnp.float32),
        scratch_types=[
            pltpu.VMEM((_CH,), jnp.int32),
            pltpu.VMEM((_CH, DIM), jnp.float32),
            pltpu.SemaphoreType.DMA,
        ],
    )
    def k(table_hbm, idx_hbm, out_hbm, idx_v, rows_v, sem):
        wid = lax.axis_index("s") * _NC + lax.axis_index("c")
        base = wid * _BPW
        for c in range(_NCH):
            off = base + c * _CH
            pltpu.sync_copy(idx_hbm.at[pl.ds(off, _CH)], idx_v)
            pltpu.async_copy(table_hbm.at[idx_v], rows_v, sem).wait()
            pltpu.sync_copy(rows_v, out_hbm.at[pl.ds(off, _CH)])

    return k(W, idx)


def kernel(z, W):
    idx3, loss2 = _dist_argmin(z, W)
    idx = idx3.reshape(N_ROWS)
    z_q = _sc_gather(W, idx)
    loss = loss2[0, 0]
    return (loss, z_q, idx)


# speedup vs baseline: 1.3085x; 1.3085x over previous
import functools

import jax
import jax.numpy as jnp
from jax import lax
from jax.experimental import pallas as pl
from jax.experimental.pallas import tpu as pltpu
from jax.experimental.pallas import tpu_sc as plsc

N_ROWS = 16384
N_CODES = 8192
DIM = 256
BETA_ = 1.0

BZ = 512
BW = 1024
NZ = N_ROWS // BZ
NWB = N_CODES // BW
LOSS_SCALE = (1.0 + BETA_) / (N_ROWS * DIM)


_LANES = 128
_NCOL = BW // _LANES


def _dist_argmin_body(z_ref, w_ref, idx_ref, loss_ref, sval, scode,
                      wn_s, t_s, w2_s):
    i = pl.program_id(0)
    j = pl.program_id(1)

    @pl.when((i == 0) & (j == 0))
    def _():
        w = w_ref[...]
        w2_s[...] = w + w
        wn_s[...] = jnp.sum(w * w, axis=1).reshape(1, N_CODES)

    @pl.when(j == 0)
    def _():
        z0 = z_ref[...]
        zn = jnp.sum(z0 * z0, axis=1, keepdims=True)
        t_s[...] = zn + wn_s[...]

    mm2 = lax.dot_general(z_ref[...], w2_s[pl.ds(j * BW, BW), :],
                          dimension_numbers=(((1,), (1,)), ((), ())),
                          preferred_element_type=jnp.float32)
    dd = t_s[:, pl.ds(j * BW, BW)] - mm2

    lane = lax.broadcasted_iota(jnp.int32, (BZ, _LANES), 1)
    cur = [(dd[:, k * _LANES:(k + 1) * _LANES], lane + (j * BW + k * _LANES))
           for k in range(_NCOL)]
    while len(cur) > 1:
        nxt = []
        for k in range(0, len(cur), 2):
            (va, ca), (vb, cb) = cur[k], cur[k + 1]
            upd = vb < va
            nxt.append((jnp.where(upd, vb, va), jnp.where(upd, cb, ca)))
        cur = nxt
    fv, fc = cur[0]

    @pl.when(j == 0)
    def _():
        sval[...] = fv
        scode[...] = fc

    @pl.when(j > 0)
    def _():
        sv = sval[...]
        upd = fv < sv
        sval[...] = jnp.where(upd, fv, sv)
        scode[...] = jnp.where(upd, fc, scode[...])

    @pl.when(j == NWB - 1)
    def _():
        sv = sval[...]
        gmin = jnp.min(sv, axis=1)
        idx_ref[0, 0, :] = jnp.min(
            jnp.where(sv == gmin[:, None], scode[...], jnp.int32(2**31 - 1)),
            axis=1)
        part = jnp.sum(gmin).reshape(1, 1)
        prev = jnp.where(i == 0, jnp.zeros((1, 1), jnp.float32), loss_ref[...])
        tot = prev + part
        loss_ref[...] = jnp.where(i == NZ - 1, tot * LOSS_SCALE, tot)


def _dist_argmin(z, W):
    return pl.pallas_call(
        _dist_argmin_body,
        grid=(NZ, NWB),
        in_specs=[
            pl.BlockSpec((BZ, DIM), lambda i, j: (i, 0)),
            pl.BlockSpec((N_CODES, DIM), lambda i, j: (0, 0)),
        ],
        out_specs=[
            pl.BlockSpec((1, 1, BZ), lambda i, j: (i, 0, 0)),
            pl.BlockSpec((1, 1), lambda i, j: (0, 0)),
        ],
        out_shape=[
            jax.ShapeDtypeStruct((NZ, 1, BZ), jnp.int32),
            jax.ShapeDtypeStruct((1, 1), jnp.float32),
        ],
        scratch_shapes=[
            pltpu.VMEM((BZ, _LANES), jnp.float32),
            pltpu.VMEM((BZ, _LANES), jnp.int32),
            pltpu.VMEM((1, N_CODES), jnp.float32),
            pltpu.VMEM((BZ, N_CODES), jnp.float32),
            pltpu.VMEM((N_CODES, DIM), jnp.float32),
        ],
        compiler_params=pltpu.CompilerParams(
            dimension_semantics=("arbitrary", "arbitrary")),
    )(z, W)


_NC = 2
_NS = 16
_NWK = _NC * _NS
_BPW = N_ROWS // _NWK
_CH = 128
_NCH = _BPW // _CH


def _sc_gather(W, idx):
    mesh = plsc.VectorSubcoreMesh(core_axis_name="c", subcore_axis_name="s")

    @functools.partial(
        pl.kernel, mesh=mesh,
        out_type=jax.ShapeDtypeStruct((N_ROWS, DIM), jnp.float32),
        scratch_types=[
            pltpu.VMEM((_CH,), jnp.int32),
            pltpu.VMEM((_CH, DIM), jnp.float32),
            pltpu.SemaphoreType.DMA,
        ],
    )
    def k(table_hbm, idx_hbm, out_hbm, idx_v, rows_v, sem):
        wid = lax.axis_index("s") * _NC + lax.axis_index("c")
        base = wid * _BPW
        for c in range(_NCH):
            off = base + c * _CH
            pltpu.sync_copy(idx_hbm.at[pl.ds(off, _CH)], idx_v)
            pltpu.async_copy(table_hbm.at[idx_v], rows_v, sem).wait()
            pltpu.sync_copy(rows_v, out_hbm.at[pl.ds(off, _CH)])

    return k(W, idx)


def kernel(z, W):
    idx3, loss2 = _dist_argmin(z, W)
    idx = idx3.reshape(N_ROWS)
    z_q = _sc_gather(W, idx)
    loss = loss2[0, 0]
    return (loss, z_q, idx)


# speedup vs baseline: 1.3171x; 1.0066x over previous
import functools

import jax
import jax.numpy as jnp
from jax import lax
from jax.experimental import pallas as pl
from jax.experimental.pallas import tpu as pltpu
from jax.experimental.pallas import tpu_sc as plsc

N_ROWS = 16384
N_CODES = 8192
DIM = 256
BETA_ = 1.0

BZ = 512
BW = 1024
NZ = N_ROWS // BZ
NWB = N_CODES // BW
LOSS_SCALE = (1.0 + BETA_) / (N_ROWS * DIM)


_LANES = 128
_NCOL = BW // _LANES


def _dist_argmin_body(z_ref, w_ref, idx_ref, loss_ref, sval, scode,
                      wn_s, t_s, w2_s):
    i = pl.program_id(0)
    j = pl.program_id(1)

    @pl.when((i == 0) & (j == 0))
    def _():
        w = w_ref[...]
        w2_s[...] = w + w
        wn_s[...] = jnp.sum(w * w, axis=1).reshape(1, N_CODES)

    @pl.when(j == 0)
    def _():
        z0 = z_ref[...]
        zn = jnp.sum(z0 * z0, axis=1, keepdims=True)
        t_s[...] = zn + wn_s[...]

    joff = pl.multiple_of(j * BW, BW)
    mm2 = lax.dot_general(z_ref[...], w2_s[pl.ds(joff, BW), :],
                          dimension_numbers=(((1,), (1,)), ((), ())),
                          preferred_element_type=jnp.float32)
    dd = t_s[:, pl.ds(joff, BW)] - mm2

    lane = lax.broadcasted_iota(jnp.int32, (BZ, _LANES), 1)
    cur = [(dd[:, k * _LANES:(k + 1) * _LANES], lane + (j * BW + k * _LANES))
           for k in range(_NCOL)]
    while len(cur) > 1:
        nxt = []
        for k in range(0, len(cur), 2):
            (va, ca), (vb, cb) = cur[k], cur[k + 1]
            upd = vb < va
            nxt.append((jnp.where(upd, vb, va), jnp.where(upd, cb, ca)))
        cur = nxt
    fv, fc = cur[0]

    @pl.when(j == 0)
    def _():
        sval[...] = fv
        scode[...] = fc

    @pl.when(j > 0)
    def _():
        sv = sval[...]
        upd = fv < sv
        sval[...] = jnp.where(upd, fv, sv)
        scode[...] = jnp.where(upd, fc, scode[...])

    @pl.when(j == NWB - 1)
    def _():
        sv = sval[...]
        gmin = jnp.min(sv, axis=1)
        idx_ref[0, 0, :] = jnp.min(
            jnp.where(sv == gmin[:, None], scode[...], jnp.int32(2**31 - 1)),
            axis=1)
        part = jnp.sum(gmin).reshape(1, 1)
        prev = jnp.where(i == 0, jnp.zeros((1, 1), jnp.float32), loss_ref[...])
        tot = prev + part
        loss_ref[...] = jnp.where(i == NZ - 1, tot * LOSS_SCALE, tot)


def _dist_argmin(z, W):
    return pl.pallas_call(
        _dist_argmin_body,
        grid=(NZ, NWB),
        in_specs=[
            pl.BlockSpec((BZ, DIM), lambda i, j: (i, 0)),
            pl.BlockSpec((N_CODES, DIM), lambda i, j: (0, 0)),
        ],
        out_specs=[
            pl.BlockSpec((1, 1, BZ), lambda i, j: (i, 0, 0)),
            pl.BlockSpec((1, 1), lambda i, j: (0, 0)),
        ],
        out_shape=[
            jax.ShapeDtypeStruct((NZ, 1, BZ), jnp.int32),
            jax.ShapeDtypeStruct((1, 1), jnp.float32),
        ],
        scratch_shapes=[
            pltpu.VMEM((BZ, _LANES), jnp.float32),
            pltpu.VMEM((BZ, _LANES), jnp.int32),
            pltpu.VMEM((1, N_CODES), jnp.float32),
            pltpu.VMEM((BZ, N_CODES), jnp.float32),
            pltpu.VMEM((N_CODES, DIM), jnp.float32),
        ],
        compiler_params=pltpu.CompilerParams(
            dimension_semantics=("arbitrary", "arbitrary")),
    )(z, W)


_NC = 2
_NS = 16
_NWK = _NC * _NS
_BPW = N_ROWS // _NWK
_CH = 128
_NCH = _BPW // _CH


def _sc_gather(W, idx):
    mesh = plsc.VectorSubcoreMesh(core_axis_name="c", subcore_axis_name="s")

    @functools.partial(
        pl.kernel, mesh=mesh,
        out_type=jax.ShapeDtypeStruct((N_ROWS, DIM), jnp.float32),
        scratch_types=[
            pltpu.VMEM((_CH,), jnp.int32),
            pltpu.VMEM((_CH, DIM), jnp.float32),
            pltpu.SemaphoreType.DMA,
        ],
    )
    def k(table_hbm, idx_hbm, out_hbm, idx_v, rows_v, sem):
        wid = lax.axis_index("s") * _NC + lax.axis_index("c")
        base = wid * _BPW
        for c in range(_NCH):
            off = base + c * _CH
            pltpu.sync_copy(idx_hbm.at[pl.ds(off, _CH)], idx_v)
            pltpu.async_copy(table_hbm.at[idx_v], rows_v, sem).wait()
            pltpu.sync_copy(rows_v, out_hbm.at[pl.ds(off, _CH)])

    return k(W, idx)


def kernel(z, W):
    idx3, loss2 = _dist_argmin(z, W)
    idx = idx3.reshape(N_ROWS)
    z_q = _sc_gather(W, idx)
    loss = loss2[0, 0]
    return (loss, z_q, idx)


# speedup vs baseline: 1.5711x; 1.1929x over previous
import functools

import jax
import jax.numpy as jnp
from jax import lax
from jax.experimental import pallas as pl
from jax.experimental.pallas import tpu as pltpu
from jax.experimental.pallas import tpu_sc as plsc

N_ROWS = 16384
N_CODES = 8192
DIM = 256
BETA_ = 1.0

BZ = 512
BW = 2048
NZ = N_ROWS // BZ
NWB = N_CODES // BW
LOSS_SCALE = (1.0 + BETA_) / (N_ROWS * DIM)


_LANES = 128
_NCOL = BW // _LANES


def _dist_argmin_body(z_ref, w_ref, idx_ref, loss_ref, sval, scode,
                      wn_s, t_s, w2_s):
    i = pl.program_id(0)
    j = pl.program_id(1)

    @pl.when((i == 0) & (j == 0))
    def _():
        w = w_ref[...]
        w2_s[...] = w + w
        wn_s[...] = jnp.sum(w * w, axis=1).reshape(1, N_CODES)

    @pl.when(j == 0)
    def _():
        z0 = z_ref[...]
        zn = jnp.sum(z0 * z0, axis=1, keepdims=True)
        t_s[...] = zn + wn_s[...]

    joff = pl.multiple_of(j * BW, BW)
    mm2 = lax.dot_general(z_ref[...], w2_s[pl.ds(joff, BW), :],
                          dimension_numbers=(((1,), (1,)), ((), ())),
                          preferred_element_type=jnp.float32)
    dd = t_s[:, pl.ds(joff, BW)] - mm2

    lane = lax.broadcasted_iota(jnp.int32, (BZ, _LANES), 1)
    cur = [(dd[:, k * _LANES:(k + 1) * _LANES], lane + (j * BW + k * _LANES))
           for k in range(_NCOL)]
    while len(cur) > 1:
        nxt = []
        for k in range(0, len(cur), 2):
            (va, ca), (vb, cb) = cur[k], cur[k + 1]
            upd = vb < va
            nxt.append((jnp.where(upd, vb, va), jnp.where(upd, cb, ca)))
        cur = nxt
    fv, fc = cur[0]

    @pl.when(j == 0)
    def _():
        sval[...] = fv
        scode[...] = fc

    @pl.when(j > 0)
    def _():
        sv = sval[...]
        upd = fv < sv
        sval[...] = jnp.where(upd, fv, sv)
        scode[...] = jnp.where(upd, fc, scode[...])

    @pl.when(j == NWB - 1)
    def _():
        sv = sval[...]
        gmin = jnp.min(sv, axis=1)
        idx_ref[0, 0, :] = jnp.min(
            jnp.where(sv == gmin[:, None], scode[...], jnp.int32(2**31 - 1)),
            axis=1)
        part = jnp.sum(gmin).reshape(1, 1)
        prev = jnp.where(i == 0, jnp.zeros((1, 1), jnp.float32), loss_ref[...])
        tot = prev + part
        loss_ref[...] = jnp.where(i == NZ - 1, tot * LOSS_SCALE, tot)


def _dist_argmin(z, W):
    return pl.pallas_call(
        _dist_argmin_body,
        grid=(NZ, NWB),
        in_specs=[
            pl.BlockSpec((BZ, DIM), lambda i, j: (i, 0)),
            pl.BlockSpec((N_CODES, DIM), lambda i, j: (0, 0)),
        ],
        out_specs=[
            pl.BlockSpec((1, 1, BZ), lambda i, j: (i, 0, 0)),
            pl.BlockSpec((1, 1), lambda i, j: (0, 0)),
        ],
        out_shape=[
            jax.ShapeDtypeStruct((NZ, 1, BZ), jnp.int32),
            jax.ShapeDtypeStruct((1, 1), jnp.float32),
        ],
        scratch_shapes=[
            pltpu.VMEM((BZ, _LANES), jnp.float32),
            pltpu.VMEM((BZ, _LANES), jnp.int32),
            pltpu.VMEM((1, N_CODES), jnp.float32),
            pltpu.VMEM((BZ, N_CODES), jnp.float32),
            pltpu.VMEM((N_CODES, DIM), jnp.float32),
        ],
        compiler_params=pltpu.CompilerParams(
            dimension_semantics=("arbitrary", "arbitrary")),
    )(z, W)


_NC = 2
_NS = 16
_NWK = _NC * _NS
_BPW = N_ROWS // _NWK
_CH = 128
_NCH = _BPW // _CH


def _sc_gather(W, idx):
    mesh = plsc.VectorSubcoreMesh(core_axis_name="c", subcore_axis_name="s")

    @functools.partial(
        pl.kernel, mesh=mesh,
        out_type=jax.ShapeDtypeStruct((N_ROWS, DIM), jnp.float32),
        scratch_types=[
            pltpu.VMEM((_CH,), jnp.int32),
            pltpu.VMEM((_CH, DIM), jnp.float32),
            pltpu.SemaphoreType.DMA,
        ],
    )
    def k(table_hbm, idx_hbm, out_hbm, idx_v, rows_v, sem):
        wid = lax.axis_index("s") * _NC + lax.axis_index("c")
        base = wid * _BPW
        for c in range(_NCH):
            off = base + c * _CH
            pltpu.sync_copy(idx_hbm.at[pl.ds(off, _CH)], idx_v)
            pltpu.async_copy(table_hbm.at[idx_v], rows_v, sem).wait()
            pltpu.sync_copy(rows_v, out_hbm.at[pl.ds(off, _CH)])

    return k(W, idx)


def kernel(z, W):
    idx3, loss2 = _dist_argmin(z, W)
    idx = idx3.reshape(N_ROWS)
    z_q = _sc_gather(W, idx)
    loss = loss2[0, 0]
    return (loss, z_q, idx)


# speedup vs baseline: 1.7668x; 1.1245x over previous
import functools

import jax
import jax.numpy as jnp
from jax import lax
from jax.experimental import pallas as pl
from jax.experimental.pallas import tpu as pltpu
from jax.experimental.pallas import tpu_sc as plsc

N_ROWS = 16384
N_CODES = 8192
DIM = 256
BETA_ = 1.0

BZ = 512
BW = 4096
NZ = N_ROWS // BZ
NWB = N_CODES // BW
LOSS_SCALE = (1.0 + BETA_) / (N_ROWS * DIM)


_LANES = 128
_NCOL = BW // _LANES


def _dist_argmin_body(z_ref, w_ref, idx_ref, loss_ref, sval, scode,
                      wn_s, t_s, w2_s):
    i = pl.program_id(0)
    j = pl.program_id(1)

    @pl.when((i == 0) & (j == 0))
    def _():
        w = w_ref[...]
        w2_s[...] = w + w
        wn_s[...] = jnp.sum(w * w, axis=1).reshape(1, N_CODES)

    @pl.when(j == 0)
    def _():
        z0 = z_ref[...]
        zn = jnp.sum(z0 * z0, axis=1, keepdims=True)
        t_s[...] = zn + wn_s[...]

    joff = pl.multiple_of(j * BW, BW)
    mm2 = lax.dot_general(z_ref[...], w2_s[pl.ds(joff, BW), :],
                          dimension_numbers=(((1,), (1,)), ((), ())),
                          preferred_element_type=jnp.float32)
    dd = t_s[:, pl.ds(joff, BW)] - mm2

    lane = lax.broadcasted_iota(jnp.int32, (BZ, _LANES), 1)
    cur = [(dd[:, k * _LANES:(k + 1) * _LANES], lane + (j * BW + k * _LANES))
           for k in range(_NCOL)]
    while len(cur) > 1:
        nxt = []
        for k in range(0, len(cur), 2):
            (va, ca), (vb, cb) = cur[k], cur[k + 1]
            upd = vb < va
            nxt.append((jnp.where(upd, vb, va), jnp.where(upd, cb, ca)))
        cur = nxt
    fv, fc = cur[0]

    @pl.when(j == 0)
    def _():
        sval[...] = fv
        scode[...] = fc

    @pl.when(j > 0)
    def _():
        sv = sval[...]
        upd = fv < sv
        sval[...] = jnp.where(upd, fv, sv)
        scode[...] = jnp.where(upd, fc, scode[...])

    @pl.when(j == NWB - 1)
    def _():
        sv = sval[...]
        gmin = jnp.min(sv, axis=1)
        idx_ref[0, 0, :] = jnp.min(
            jnp.where(sv == gmin[:, None], scode[...], jnp.int32(2**31 - 1)),
            axis=1)
        part = jnp.sum(gmin).reshape(1, 1)
        prev = jnp.where(i == 0, jnp.zeros((1, 1), jnp.float32), loss_ref[...])
        tot = prev + part
        loss_ref[...] = jnp.where(i == NZ - 1, tot * LOSS_SCALE, tot)


def _dist_argmin(z, W):
    return pl.pallas_call(
        _dist_argmin_body,
        grid=(NZ, NWB),
        in_specs=[
            pl.BlockSpec((BZ, DIM), lambda i, j: (i, 0)),
            pl.BlockSpec((N_CODES, DIM), lambda i, j: (0, 0)),
        ],
        out_specs=[
            pl.BlockSpec((1, 1, BZ), lambda i, j: (i, 0, 0)),
            pl.BlockSpec((1, 1), lambda i, j: (0, 0)),
        ],
        out_shape=[
            jax.ShapeDtypeStruct((NZ, 1, BZ), jnp.int32),
            jax.ShapeDtypeStruct((1, 1), jnp.float32),
        ],
        scratch_shapes=[
            pltpu.VMEM((BZ, _LANES), jnp.float32),
            pltpu.VMEM((BZ, _LANES), jnp.int32),
            pltpu.VMEM((1, N_CODES), jnp.float32),
            pltpu.VMEM((BZ, N_CODES), jnp.float32),
            pltpu.VMEM((N_CODES, DIM), jnp.float32),
        ],
        compiler_params=pltpu.CompilerParams(
            dimension_semantics=("arbitrary", "arbitrary")),
    )(z, W)


_NC = 2
_NS = 16
_NWK = _NC * _NS
_BPW = N_ROWS // _NWK
_CH = 128
_NCH = _BPW // _CH


def _sc_gather(W, idx):
    mesh = plsc.VectorSubcoreMesh(core_axis_name="c", subcore_axis_name="s")

    @functools.partial(
        pl.kernel, mesh=mesh,
        out_type=jax.ShapeDtypeStruct((N_ROWS, DIM), jnp.float32),
        scratch_types=[
            pltpu.VMEM((_CH,), jnp.int32),
            pltpu.VMEM((_CH, DIM), jnp.float32),
            pltpu.SemaphoreType.DMA,
        ],
    )
    def k(table_hbm, idx_hbm, out_hbm, idx_v, rows_v, sem):
        wid = lax.axis_index("s") * _NC + lax.axis_index("c")
        base = wid * _BPW
        for c in range(_NCH):
            off = base + c * _CH
            pltpu.sync_copy(idx_hbm.at[pl.ds(off, _CH)], idx_v)
            pltpu.async_copy(table_hbm.at[idx_v], rows_v, sem).wait()
            pltpu.sync_copy(rows_v, out_hbm.at[pl.ds(off, _CH)])

    return k(W, idx)


def kernel(z, W):
    idx3, loss2 = _dist_argmin(z, W)
    idx = idx3.reshape(N_ROWS)
    z_q = _sc_gather(W, idx)
    loss = loss2[0, 0]
    return (loss, z_q, idx)


# speedup vs baseline: 2.0138x; 1.1398x over previous
import functools

import jax
import jax.numpy as jnp
from jax import lax
from jax.experimental import pallas as pl
from jax.experimental.pallas import tpu as pltpu
from jax.experimental.pallas import tpu_sc as plsc

N_ROWS = 16384
N_CODES = 8192
DIM = 256
BETA_ = 1.0

BZ = 512
BW = 8192
NZ = N_ROWS // BZ
NWB = N_CODES // BW
LOSS_SCALE = (1.0 + BETA_) / (N_ROWS * DIM)


_LANES = 128
_NCOL = BW // _LANES


def _dist_argmin_body(z_ref, w_ref, idx_ref, loss_ref, sval, scode,
                      wn_s, t_s, w2_s):
    i = pl.program_id(0)
    j = pl.program_id(1)

    @pl.when((i == 0) & (j == 0))
    def _():
        w = w_ref[...]
        w2_s[...] = w + w
        wn_s[...] = jnp.sum(w * w, axis=1).reshape(1, N_CODES)

    @pl.when(j == 0)
    def _():
        z0 = z_ref[...]
        zn = jnp.sum(z0 * z0, axis=1, keepdims=True)
        t_s[...] = zn + wn_s[...]

    joff = pl.multiple_of(j * BW, BW)
    mm2 = lax.dot_general(z_ref[...], w2_s[pl.ds(joff, BW), :],
                          dimension_numbers=(((1,), (1,)), ((), ())),
                          preferred_element_type=jnp.float32)
    dd = t_s[:, pl.ds(joff, BW)] - mm2

    lane = lax.broadcasted_iota(jnp.int32, (BZ, _LANES), 1)
    cur = [(dd[:, k * _LANES:(k + 1) * _LANES], lane + (j * BW + k * _LANES))
           for k in range(_NCOL)]
    while len(cur) > 1:
        nxt = []
        for k in range(0, len(cur), 2):
            (va, ca), (vb, cb) = cur[k], cur[k + 1]
            upd = vb < va
            nxt.append((jnp.where(upd, vb, va), jnp.where(upd, cb, ca)))
        cur = nxt
    fv, fc = cur[0]

    @pl.when(j == 0)
    def _():
        sval[...] = fv
        scode[...] = fc

    @pl.when(j > 0)
    def _():
        sv = sval[...]
        upd = fv < sv
        sval[...] = jnp.where(upd, fv, sv)
        scode[...] = jnp.where(upd, fc, scode[...])

    @pl.when(j == NWB - 1)
    def _():
        sv = sval[...]
        gmin = jnp.min(sv, axis=1)
        idx_ref[0, 0, :] = jnp.min(
            jnp.where(sv == gmin[:, None], scode[...], jnp.int32(2**31 - 1)),
            axis=1)
        part = jnp.sum(gmin).reshape(1, 1)
        prev = jnp.where(i == 0, jnp.zeros((1, 1), jnp.float32), loss_ref[...])
        tot = prev + part
        loss_ref[...] = jnp.where(i == NZ - 1, tot * LOSS_SCALE, tot)


def _dist_argmin(z, W):
    return pl.pallas_call(
        _dist_argmin_body,
        grid=(NZ, NWB),
        in_specs=[
            pl.BlockSpec((BZ, DIM), lambda i, j: (i, 0)),
            pl.BlockSpec((N_CODES, DIM), lambda i, j: (0, 0)),
        ],
        out_specs=[
            pl.BlockSpec((1, 1, BZ), lambda i, j: (i, 0, 0)),
            pl.BlockSpec((1, 1), lambda i, j: (0, 0)),
        ],
        out_shape=[
            jax.ShapeDtypeStruct((NZ, 1, BZ), jnp.int32),
            jax.ShapeDtypeStruct((1, 1), jnp.float32),
        ],
        scratch_shapes=[
            pltpu.VMEM((BZ, _LANES), jnp.float32),
            pltpu.VMEM((BZ, _LANES), jnp.int32),
            pltpu.VMEM((1, N_CODES), jnp.float32),
            pltpu.VMEM((BZ, N_CODES), jnp.float32),
            pltpu.VMEM((N_CODES, DIM), jnp.float32),
        ],
        compiler_params=pltpu.CompilerParams(
            dimension_semantics=("arbitrary", "arbitrary")),
    )(z, W)


_NC = 2
_NS = 16
_NWK = _NC * _NS
_BPW = N_ROWS // _NWK
_CH = 128
_NCH = _BPW // _CH


def _sc_gather(W, idx):
    mesh = plsc.VectorSubcoreMesh(core_axis_name="c", subcore_axis_name="s")

    @functools.partial(
        pl.kernel, mesh=mesh,
        out_type=jax.ShapeDtypeStruct((N_ROWS, DIM), jnp.float32),
        scratch_types=[
            pltpu.VMEM((_CH,), jnp.int32),
            pltpu.VMEM((_CH, DIM), jnp.float32),
            pltpu.SemaphoreType.DMA,
        ],
    )
    def k(table_hbm, idx_hbm, out_hbm, idx_v, rows_v, sem):
        wid = lax.axis_index("s") * _NC + lax.axis_index("c")
        base = wid * _BPW
        for c in range(_NCH):
            off = base + c * _CH
            pltpu.sync_copy(idx_hbm.at[pl.ds(off, _CH)], idx_v)
            pltpu.async_copy(table_hbm.at[idx_v], rows_v, sem).wait()
            pltpu.sync_copy(rows_v, out_hbm.at[pl.ds(off, _CH)])

    return k(W, idx)


def kernel(z, W):
    idx3, loss2 = _dist_argmin(z, W)
    idx = idx3.reshape(N_ROWS)
    z_q = _sc_gather(W, idx)
    loss = loss2[0, 0]
    return (loss, z_q, idx)
